# Initial kernel scaffold; baseline (speedup 1.0000x reference)
import jax
import jax.numpy as jnp
from jax.experimental import pallas as pl


def kernel(x, edge_index, edge_attr, batch, atom_emb, bond_emb, eps, W1, b1, g1, be1, W2, b2, g2, be2, Wp, bp):
    raise NotImplementedError("write your pallas kernel here")


# speedup vs baseline: 2.7131x; 2.7131x over previous
import functools

import jax

jax.config.update("jax_default_matmul_precision", "highest")
import jax.numpy as jnp
from jax import lax
from jax.experimental import pallas as pl
from jax.experimental.pallas import tpu as pltpu
from jax.experimental.pallas import tpu_sc as plsc

_N = 10000
_E = 320000
_D = 128
_L = 5
_G = 64

_NC, _NS = 2, 16
_NW = _NC * _NS
_NPAD = 10240
_ROWS_W = _NPAD // _NW
_ROWS_T = _NPAD // _NS
_CN = 80
_CE = 128
_EPAD = 327680
_EW = _EPAD // _NW
_NCH = _EW // _CE

_mesh = plsc.VectorSubcoreMesh(core_axis_name="c", subcore_axis_name="s")


def _atom_body(xT, tab, out, idxraw, idxbuf, gbuf, acc, sem):
    c = lax.axis_index("c")
    s = lax.axis_index("s")
    w = c * _NS + s
    base = w * _ROWS_W

    def chunk(ci, _):
        b = base + ci * _CN
        for i in range(9):
            pltpu.sync_copy(xT.at[pl.ds(i * _NPAD + b, _CN)], idxraw)

            def addoff(j, _):
                sl = pl.ds(j * 16, 16)
                idxbuf[sl] = idxraw[sl] + (i * 100)
                return 0

            lax.fori_loop(0, _CN // 16, addoff, 0)
            dst = acc if i == 0 else gbuf
            pltpu.async_copy(tab.at[idxbuf], dst, sem).wait()
            if i > 0:
                def accrow(r, _):
                    for k in range(8):
                        sl = pl.ds(k * 16, 16)
                        acc[r, sl] = acc[r, sl] + gbuf[r, sl]
                    return 0

                lax.fori_loop(0, _CN, accrow, 0)
        pltpu.sync_copy(acc, out.at[pl.ds(b, _CN)])
        return 0

    lax.fori_loop(0, _ROWS_W // _CN, chunk, 0)


_atom_call = pl.kernel(
    _atom_body,
    out_type=jax.ShapeDtypeStruct((_NPAD, _D), jnp.float32),
    mesh=_mesh,
    scratch_types=[
        pltpu.VMEM((_CN,), jnp.int32),
        pltpu.VMEM((_CN,), jnp.int32),
        pltpu.VMEM((_CN, _D), jnp.float32),
        pltpu.VMEM((_CN, _D), jnp.float32),
        pltpu.SemaphoreType.DMA,
    ],
)


def _edge_body(hmat, bt, srcP, dstP, eaT, out,
               ibs, ibd, eab, cb, hbuf, ebuf, aggr, sem):
    c = lax.axis_index("c")
    s = lax.axis_index("s")

    def zrow(r, _):
        for k in range(8):
            hbuf[r, pl.ds(k * 16, 16)] = jnp.zeros((16,), jnp.float32)
        return 0

    lax.fori_loop(0, _CE, zrow, 0)
    zbase = s * _ROWS_T

    def zcopy(q, _):
        pltpu.sync_copy(hbuf, aggr.at[pl.ds(zbase + q * _CE, _CE)])
        return 0

    lax.fori_loop(0, _ROWS_T // _CE, zcopy, 0)
    plsc.subcore_barrier()

    w = c * _NS + s
    ebase = w * _EW

    def chunk(ci, _):
        b = ebase + ci * _CE
        pltpu.sync_copy(srcP.at[pl.ds(b, _CE)], ibs)
        pltpu.sync_copy(dstP.at[pl.ds(b, _CE)], ibd)
        for j in range(3):
            pltpu.sync_copy(eaT.at[pl.ds(j * _EPAD + b, _CE)], eab.at[j])

        def codev(j, _):
            sl = pl.ds(j * 16, 16)
            cb[sl] = eab[0, sl] * 64 + eab[1, sl] * 8 + eab[2, sl]
            return 0

        lax.fori_loop(0, _CE // 16, codev, 0)
        d1 = pltpu.async_copy(hmat.at[ibs], hbuf, sem)
        d2 = pltpu.async_copy(bt.at[cb], ebuf, sem)
        d1.wait()
        d2.wait()

        def mrow(r, _):
            for k in range(8):
                sl = pl.ds(k * 16, 16)
                hbuf[r, sl] = jnp.maximum(hbuf[r, sl] + ebuf[r, sl], 0.0)
            return 0

        lax.fori_loop(0, _CE, mrow, 0)
        pltpu.sync_copy(hbuf, aggr.at[ibd], add=True)
        return 0

    lax.fori_loop(0, _NCH, chunk, 0)
    plsc.subcore_barrier()

    obase = s * _ROWS_T

    def ocopy(q, _):
        r0 = obase + q * _CE
        pltpu.sync_copy(aggr.at[pl.ds(r0, _CE)], out.at[c, pl.ds(r0, _CE)])
        return 0

    lax.fori_loop(0, _ROWS_T // _CE, ocopy, 0)


_edge_call = pl.kernel(
    _edge_body,
    out_type=jax.ShapeDtypeStruct((_NC, _NPAD, _D), jnp.float32),
    mesh=_mesh,
    scratch_types=[
        pltpu.VMEM((_CE,), jnp.int32),
        pltpu.VMEM((_CE,), jnp.int32),
        pltpu.VMEM((3, _CE), jnp.int32),
        pltpu.VMEM((_CE,), jnp.int32),
        pltpu.VMEM((_CE, _D), jnp.float32),
        pltpu.VMEM((_CE, _D), jnp.float32),
        pltpu.VMEM_SHARED((_NPAD, _D), jnp.float32),
        pltpu.SemaphoreType.DMA,
    ],
)

_PREC = lax.Precision.HIGHEST
_EXACT = lax.Precision.HIGHEST


def _bn(t, mask, g, b):
    m = jnp.sum(t * mask, axis=0, keepdims=True) * (1.0 / _N)
    v = jnp.sum(((t - m) ** 2) * mask, axis=0, keepdims=True) * (1.0 / _N)
    return (t - m) * lax.rsqrt(v + 1e-5) * g + b


def _mlp_body(h, pa, eps, msk, W1, b1, g1, be1, W2, b2, g2, be2, out):
    mask = msk[...]
    hin = (1.0 + eps[0, 0]) * h[...] + pa[...]
    t = lax.dot_general(hin, W1[...], (((1,), (0,)), ((), ())),
                        precision=_PREC) + b1[...]
    t = jnp.maximum(_bn(t, mask, g1[...], be1[...]), 0.0)
    t = lax.dot_general(t, W2[...], (((1,), (0,)), ((), ())),
                        precision=_PREC) + b2[...]
    t = _bn(t, mask, g2[...], be2[...])
    out[...] = jnp.maximum(t, 0.0)


def _mlp_pool_body(h, pa, eps, msk, W1, b1, g1, be1, W2, b2, g2, be2,
                   batv, Wp, bp, out):
    mask = msk[...]
    hin = (1.0 + eps[0, 0]) * h[...] + pa[...]
    t = lax.dot_general(hin, W1[...], (((1,), (0,)), ((), ())),
                        precision=_PREC) + b1[...]
    t = jnp.maximum(_bn(t, mask, g1[...], be1[...]), 0.0)
    t = lax.dot_general(t, W2[...], (((1,), (0,)), ((), ())),
                        precision=_PREC) + b2[...]
    t = _bn(t, mask, g2[...], be2[...])
    gid = lax.broadcasted_iota(jnp.int32, (1, _G), 1)
    oh = (batv[...] == gid).astype(jnp.float32)
    sums = lax.dot_general(oh, t, (((0,), (0,)), ((), ())), precision=_EXACT)
    cnt = lax.dot_general(oh, mask, (((0,), (0,)), ((), ())), precision=_EXACT)
    hg = sums / jnp.maximum(cnt, 1.0)
    out[...] = lax.dot_general(hg, Wp[...], (((1,), (0,)), ((), ())),
                               precision=_PREC) + bp[...]


def _mlp_call(h, p, eps_l, msk, W1l, b1l, g1l, be1l, W2l, b2l, g2l, be2l):
    return pl.pallas_call(
        _mlp_body,
        out_shape=jax.ShapeDtypeStruct((_NPAD, _D), jnp.float32),
    )(h, p, eps_l, msk, W1l, b1l, g1l, be1l, W2l, b2l, g2l, be2l)


def _mlp_pool_call(h, p, eps_l, msk, W1l, b1l, g1l, be1l, W2l, b2l, g2l,
                   be2l, batv, Wpp, bpp):
    return pl.pallas_call(
        _mlp_pool_body,
        out_shape=jax.ShapeDtypeStruct((_G, 16), jnp.float32),
    )(h, p, eps_l, msk, W1l, b1l, g1l, be1l, W2l, b2l, g2l, be2l,
      batv, Wpp, bpp)


def kernel(x, edge_index, edge_attr, batch, atom_emb, bond_emb, eps,
           W1, b1, g1, be1, W2, b2, g2, be2, Wp, bp):
    f32 = jnp.float32
    i32 = jnp.int32
    xT = jnp.pad(x.T.astype(i32), ((0, 0), (0, _NPAD - _N))).reshape(-1)
    atab = atom_emb.reshape(9 * 100, _D).astype(f32)
    srcP = jnp.pad(edge_index[0].astype(i32), (0, _EPAD - _E))
    dstP = jnp.pad(edge_index[1].astype(i32), (0, _EPAD - _E),
                   constant_values=_N)
    eaT = jnp.pad(edge_attr.T.astype(i32),
                  ((0, 0), (0, _EPAD - _E))).reshape(-1)
    BT = (bond_emb[:, 0, :, None, None, :]
          + bond_emb[:, 1, None, :, None, :]
          + bond_emb[:, 2, None, None, :, :]).reshape(_L, 512, _D)
    batv = jnp.pad(batch.astype(i32), (0, _NPAD - _N),
                   constant_values=_G).reshape(_NPAD, 1)
    Wpp = jnp.pad(Wp, ((0, 0), (0, 6)))
    bpp = jnp.pad(bp, (0, 6)).reshape(1, 16)

    rmask = (jnp.arange(_NPAD) < _N).astype(f32).reshape(_NPAD, 1)
    h = _atom_call(xT, atab)
    for l in range(_L):
        part = _edge_call(h, BT[l], srcP, dstP, eaT)
        pa = part[0] + part[1]
        args = (h, pa, eps[l].reshape(1, 1), rmask,
                W1[l], b1[l].reshape(1, -1), g1[l].reshape(1, -1),
                be1[l].reshape(1, -1),
                W2[l], b2[l].reshape(1, -1), g2[l].reshape(1, -1),
                be2[l].reshape(1, -1))
        if l < _L - 1:
            h = _mlp_call(*args)
        else:
            outp = _mlp_pool_call(*args, batv, Wpp, bpp)
    return outp[:, :10]
